# baseline (speedup 1.0000x reference)
import functools

import jax
import jax.numpy as jnp
from jax import lax
from jax.experimental import pallas as pl
from jax.experimental.pallas import tpu as pltpu
from jax.experimental.pallas import tpu_sc as plsc

_NUM_CODONS = 64
_B, _L = 4096, 2048
_W = _L // 4
_NC, _NS, _LANES = 2, 16, 16
_NW = _NC * _NS
_ROWS_PER_WORKER = _B // _NW
_RC = 16
_NCHUNK = _ROWS_PER_WORKER // _RC
_TBL = 80
_NA = 4

_HI = jnp.int32(-2139062144)
_LO7F = jnp.int32(0x7F7F7F7F)
_MUL1 = jnp.int32(0x01010101)


def _make_sc_kernel():
    mesh = plsc.VectorSubcoreMesh(core_axis_name="c", subcore_axis_name="s")

    @functools.partial(
        pl.kernel,
        mesh=mesh,
        out_type=jax.ShapeDtypeStruct((_B,), jnp.float32),
        compiler_params=pltpu.CompilerParams(needs_layout_passes=False),
        scratch_types=[
            pltpu.VMEM((_TBL * _LANES,), jnp.float32),
            pltpu.VMEM((_RC * _W,), jnp.int32),
            pltpu.VMEM((_RC * _W,), jnp.int32),
            pltpu.VMEM((_RC * 17,), jnp.float32),
            pltpu.VMEM((_RC * 17,), jnp.int32),
            pltpu.VMEM((_ROWS_PER_WORKER,), jnp.float32),
            pltpu.SemaphoreType.DMA,
            pltpu.SemaphoreType.DMA,
        ],
    )
    def sc_kernel(rna_hbm, table_hbm, out_hbm, table_v, buf0, buf1,
                  accmat, cntmat, out_v, sem0, sem1):
        wid = lax.axis_index("s") * _NC + lax.axis_index("c")
        base = wid * _ROWS_PER_WORKER

        pltpu.sync_copy(table_hbm, table_v)

        bufs = (buf0, buf1)
        sems = (sem0, sem1)
        iota = lax.iota(jnp.int32, _LANES)
        iota17 = iota * 17
        zf = jnp.zeros((_LANES,), jnp.float32)
        zi = jnp.zeros((_LANES,), jnp.int32)

        for r in range(_RC):
            pltpu.async_copy(
                rna_hbm.at[pl.ds((base + r) * _W, _W)],
                buf0.at[pl.ds(r * _W, _W)], sem0)

        for c in range(_NCHUNK):
            cur = c % 2
            buf = bufs[cur]

            def row_body(r, _, buf=buf, sem=sems[cur], c=c, cur=cur):
                pltpu.make_async_copy(
                    rna_hbm.at[pl.ds(0, _W)],
                    buf.at[pl.ds(r * _W, _W)], sem).wait()
                if c + 1 < _NCHUNK:
                    pltpu.async_copy(
                        rna_hbm.at[pl.ds((base + (c + 1) * _RC) * _W + r * _W,
                                         _W)],
                        bufs[1 - cur].at[pl.ds(r * _W, _W)], sems[1 - cur])
                row_off = r * _W

                def body(s, carry, buf=buf, row_off=row_off):
                    accs, cnts = carry
                    accs, cnts = list(accs), list(cnts)
                    off = row_off + s * (_NA * _LANES)
                    for u in range(_NA):
                        w = buf[pl.ds(off + u * _LANES, _LANES)]
                        i0 = ((w << 4) & 0xFF0) | iota
                        i1 = ((w >> 4) & 0xFF0) | iota
                        i2 = ((w >> 12) & 0xFF0) | iota
                        i3 = ((w >> 20) & 0xFF0) | iota
                        t0 = plsc.load_gather(table_v, [i0])
                        t1 = plsc.load_gather(table_v, [i1])
                        t2 = plsc.load_gather(table_v, [i2])
                        t3 = plsc.load_gather(table_v, [i3])
                        accs[u] = accs[u] + ((t0 + t1) + (t2 + t3))
                        nz = lax.shift_right_logical(
                            (w + _LO7F) & _HI, jnp.int32(7))
                        cnts[u] = cnts[u] + lax.shift_right_logical(
                            nz * _MUL1, jnp.int32(24))
                    return tuple(accs), tuple(cnts)

                accs, cnts = plsc.parallel_loop(
                    0, _W // (_NA * _LANES), 1, unroll=2,
                    carry=((zf,) * _NA, (zi,) * _NA))(body)
                acc = (accs[0] + accs[1]) + (accs[2] + accs[3])
                cnt = (cnts[0] + cnts[1]) + (cnts[2] + cnts[3])
                accmat[pl.ds(r * 17, _LANES)] = acc
                cntmat[pl.ds(r * 17, _LANES)] = cnt
                return 0

            lax.fori_loop(0, _RC, row_body, 0)

            att = zf
            ctt = zi
            for l in range(_LANES):
                att = att + plsc.load_gather(accmat, [iota17 + l])
                ctt = ctt + plsc.load_gather(cntmat, [iota17 + l])
            out_v[pl.ds(c * _RC, _RC)] = att / ctt.astype(jnp.float32)

        pltpu.sync_copy(out_v, out_hbm.at[pl.ds(base, _ROWS_PER_WORKER)])

    return sc_kernel


_SC_KERNEL = _make_sc_kernel()


def kernel(rna_data, tissue_id, W, b):
    del tissue_id
    table = jnp.zeros((_TBL,), jnp.float32).at[1:_NUM_CODONS + 1].set(W[0] + b[0])
    table_b = jnp.repeat(table, _LANES)
    packed = jax.lax.bitcast_convert_type(
        rna_data.astype(jnp.int8).reshape(_B, _W, 4), jnp.int32)
    y = _SC_KERNEL(packed.reshape(_B * _W), table_b)
    return y.reshape(_B, 1)


# speedup vs baseline: 2.5845x; 2.5845x over previous
import functools

import jax
import jax.numpy as jnp
from jax import lax
from jax.experimental import pallas as pl
from jax.experimental.pallas import tpu as pltpu
from jax.experimental.pallas import tpu_sc as plsc

_NUM_CODONS = 64
_B, _L = 4096, 2048
_W = _L // 4
_NC, _NS, _LANES = 2, 16, 16
_NW = _NC * _NS
_ROWS_PER_WORKER = _B // _NW
_RC = 16
_NCHUNK = _ROWS_PER_WORKER // _RC
_TBL = 80
_NA = 4

_HI = jnp.int32(-2139062144)
_LO7F = jnp.int32(0x7F7F7F7F)
_MUL1 = jnp.int32(0x01010101)


def _make_sc_kernel():
    mesh = plsc.VectorSubcoreMesh(core_axis_name="c", subcore_axis_name="s")

    @functools.partial(
        pl.kernel,
        mesh=mesh,
        out_type=jax.ShapeDtypeStruct((_B,), jnp.float32),
        compiler_params=pltpu.CompilerParams(needs_layout_passes=False),
        scratch_types=[
            pltpu.VMEM((_TBL * _LANES,), jnp.float32),
            pltpu.VMEM((_RC * _W,), jnp.int32),
            pltpu.VMEM((_RC * _W,), jnp.int32),
            pltpu.VMEM((_RC * 17,), jnp.float32),
            pltpu.VMEM((_RC * 17,), jnp.int32),
            pltpu.VMEM((_ROWS_PER_WORKER,), jnp.float32),
            pltpu.SemaphoreType.DMA,
            pltpu.SemaphoreType.DMA,
        ],
    )
    def sc_kernel(rna_hbm, table_hbm, out_hbm, table_v, buf0, buf1,
                  accmat, cntmat, out_v, sem0, sem1):
        wid = lax.axis_index("s") * _NC + lax.axis_index("c")
        base = wid * _ROWS_PER_WORKER

        pltpu.sync_copy(table_hbm, table_v)

        bufs = (buf0, buf1)
        sems = (sem0, sem1)
        iota = lax.iota(jnp.int32, _LANES)
        iota17 = iota * 17
        zf = jnp.zeros((_LANES,), jnp.float32)
        zi = jnp.zeros((_LANES,), jnp.int32)

        for r in range(_RC):
            pltpu.async_copy(
                rna_hbm.at[pl.ds((base + r) * _W, _W)],
                buf0.at[pl.ds(r * _W, _W)], sem0)

        for c in range(_NCHUNK):
            cur = c % 2
            buf = bufs[cur]

            def row_body(r, _, buf=buf, sem=sems[cur], c=c, cur=cur):
                pltpu.make_async_copy(
                    rna_hbm.at[pl.ds(0, _W)],
                    buf.at[pl.ds(r * _W, _W)], sem).wait()
                if c + 1 < _NCHUNK:
                    pltpu.async_copy(
                        rna_hbm.at[pl.ds((base + (c + 1) * _RC) * _W + r * _W,
                                         _W)],
                        bufs[1 - cur].at[pl.ds(r * _W, _W)], sems[1 - cur])
                row_off = r * _W

                def body(s, carry, buf=buf, row_off=row_off):
                    accs, cnts = carry
                    accs, cnts = list(accs), list(cnts)
                    off = row_off + s * (_NA * _LANES)
                    for u in range(_NA):
                        w = buf[pl.ds(off + u * _LANES, _LANES)]
                        i0 = ((w << 4) & 0xFF0) | iota
                        i1 = ((w >> 4) & 0xFF0) | iota
                        i2 = ((w >> 12) & 0xFF0) | iota
                        i3 = ((w >> 20) & 0xFF0) | iota
                        t0 = plsc.load_gather(table_v, [i0])
                        t1 = plsc.load_gather(table_v, [i1])
                        t2 = plsc.load_gather(table_v, [i2])
                        t3 = plsc.load_gather(table_v, [i3])
                        accs[u] = accs[u] + ((t0 + t1) + (t2 + t3))
                        nz = lax.shift_right_logical(
                            (w + _LO7F) & _HI, jnp.int32(7))
                        cnts[u] = cnts[u] + lax.shift_right_logical(
                            nz * _MUL1, jnp.int32(24))
                    return tuple(accs), tuple(cnts)

                accs, cnts = plsc.parallel_loop(
                    0, _W // (_NA * _LANES), 1, unroll=2,
                    carry=((zf,) * _NA, (zi,) * _NA))(body)
                acc = (accs[0] + accs[1]) + (accs[2] + accs[3])
                cnt = (cnts[0] + cnts[1]) + (cnts[2] + cnts[3])
                accmat[pl.ds(r * 17, _LANES)] = acc
                cntmat[pl.ds(r * 17, _LANES)] = cnt
                return 0

            lax.fori_loop(0, _RC, row_body, 0)

            att = zf
            ctt = zi
            for l in range(_LANES):
                att = att + plsc.load_gather(accmat, [iota17 + l])
                ctt = ctt + plsc.load_gather(cntmat, [iota17 + l])
            out_v[pl.ds(c * _RC, _RC)] = att / ctt.astype(jnp.float32)

        pltpu.sync_copy(out_v, out_hbm.at[pl.ds(base, _ROWS_PER_WORKER)])

    return sc_kernel


_SC_KERNEL = _make_sc_kernel()


def kernel(rna_data, tissue_id, W, b):
    del tissue_id
    table = jnp.zeros((_TBL,), jnp.float32).at[1:_NUM_CODONS + 1].set(W[0] + b[0])
    table_b = jnp.repeat(table, _LANES)
    x = rna_data
    packed = (x[:, 0:_W] | (x[:, _W:2 * _W] << 8)
              | (x[:, 2 * _W:3 * _W] << 16) | (x[:, 3 * _W:4 * _W] << 24))
    y = _SC_KERNEL(packed.reshape(_B * _W), table_b)
    return y.reshape(_B, 1)
